# Initial kernel scaffold; baseline (speedup 1.0000x reference)
import functools

import jax
import jax.numpy as jnp
from jax import lax
from jax.experimental import pallas as pl
from jax.experimental.pallas import tpu as pltpu
from jax.experimental.pallas import tpu_sc as plsc

N, E, D, ED, H, L = 10000, 320000, 128, 16, 128, 3
LANES = 16
NC, NS = 2, 16
NW = NC * NS
EPT = E // NW
C = 80
ITERS = EPT // C


def _proj_body(x_ref, w_ref, b_ref, o_ref):
    o_ref[...] = (
        jax.lax.dot_general(x_ref[...], w_ref[...], (((1,), (0,)), ((), ())),
                            preferred_element_type=jnp.float32,
                            precision=jax.lax.Precision.HIGHEST)
        + b_ref[...]
    )


def _proj(x, w, b, blk):
    n = x.shape[0]
    return pl.pallas_call(
        _proj_body,
        grid=(n // blk,),
        in_specs=[
            pl.BlockSpec((blk, x.shape[1]), lambda i: (i, 0)),
            pl.BlockSpec(w.shape, lambda i: (0, 0)),
            pl.BlockSpec((1, b.shape[1]), lambda i: (0, 0)),
        ],
        out_specs=pl.BlockSpec((blk, w.shape[1]), lambda i: (i, 0)),
        out_shape=jax.ShapeDtypeStruct((n, w.shape[1]), jnp.float32),
    )(x, w, b)


def _node_body(h_ref, a0_ref, a1_ref, scale_ref, w1_ref, b1_ref, w2_ref,
               b2_ref, g_ref, be_ref, o_ref):
    z = scale_ref[0, 0] * h_ref[...] + a0_ref[...] + a1_ref[...]
    t = jax.lax.dot_general(z, w1_ref[...], (((1,), (0,)), ((), ())),
                            preferred_element_type=jnp.float32,
                            precision=jax.lax.Precision.HIGHEST)
    t = jnp.maximum(t + b1_ref[...], 0.0)
    y = jax.lax.dot_general(t, w2_ref[...], (((1,), (0,)), ((), ())),
                            preferred_element_type=jnp.float32,
                            precision=jax.lax.Precision.HIGHEST)
    y = y + b2_ref[...]
    mu = jnp.mean(y, axis=-1, keepdims=True)
    var = jnp.mean((y - mu) ** 2, axis=-1, keepdims=True)
    y = (y - mu) * jax.lax.rsqrt(var + 1e-5) * g_ref[...] + be_ref[...]
    o_ref[...] = jnp.maximum(y, 0.0)


def _node_update(h, a0, a1, scale, w1, b1, w2, b2, gamma, beta, blk):
    n = h.shape[0]
    return pl.pallas_call(
        _node_body,
        grid=(n // blk,),
        in_specs=[
            pl.BlockSpec((blk, H), lambda i: (i, 0)),
            pl.BlockSpec((blk, H), lambda i: (i, 0)),
            pl.BlockSpec((blk, H), lambda i: (i, 0)),
            pl.BlockSpec(memory_space=pltpu.SMEM),
            pl.BlockSpec((H, 2 * H), lambda i: (0, 0)),
            pl.BlockSpec((1, 2 * H), lambda i: (0, 0)),
            pl.BlockSpec((2 * H, H), lambda i: (0, 0)),
            pl.BlockSpec((1, H), lambda i: (0, 0)),
            pl.BlockSpec((1, H), lambda i: (0, 0)),
            pl.BlockSpec((1, H), lambda i: (0, 0)),
        ],
        out_specs=pl.BlockSpec((blk, H), lambda i: (i, 0)),
        out_shape=jax.ShapeDtypeStruct((n, H), jnp.float32),
    )(h, a0, a1, scale, w1, b1, w2, b2, gamma, beta)


@functools.partial(
    pl.kernel,
    out_type=jax.ShapeDtypeStruct((NC, N, H), jnp.float32),
    mesh=plsc.VectorSubcoreMesh(core_axis_name="c", subcore_axis_name="s"),
    scratch_types=[
        pltpu.VMEM((C,), jnp.int32),
        pltpu.VMEM((C,), jnp.int32),
        pltpu.VMEM((C, H), jnp.float32),
        pltpu.VMEM((C, H), jnp.float32),
        pltpu.VMEM_SHARED((N, H), jnp.float32),
        pltpu.SemaphoreType.DMA,
        pltpu.SemaphoreType.DMA,
    ],
)
def _agg(e_hbm, h_hbm, src_hbm, dst_hbm, zeros_hbm, out_hbm,
         src_v, dst_v, e_v, g_v, agg_sh, sem_e, sem_g):
    c = lax.axis_index("c")
    s = lax.axis_index("s")
    wid = s * NC + c

    pltpu.sync_copy(zeros_hbm, agg_sh.at[pl.ds(s * (N // NS), N // NS)])
    plsc.subcore_barrier()

    def chunk(it, carry):
        base = wid * EPT + it * C
        pltpu.sync_copy(src_hbm.at[pl.ds(base, C)], src_v)
        pltpu.sync_copy(dst_hbm.at[pl.ds(base, C)], dst_v)
        cp_e = pltpu.async_copy(e_hbm.at[pl.ds(base, C)], e_v, sem_e)
        cp_g = pltpu.async_copy(h_hbm.at[src_v], g_v, sem_g)
        cp_e.wait()
        cp_g.wait()

        def row(i, carry2):
            for j in range(H // LANES):
                sl = pl.ds(j * LANES, LANES)
                g_v[i, sl] = jnp.maximum(g_v[i, sl] + e_v[i, sl], 0.0)
            return carry2

        lax.fori_loop(0, C, row, 0)
        pltpu.sync_copy(g_v, agg_sh.at[dst_v], add=True)
        return carry

    lax.fori_loop(0, ITERS, chunk, 0)
    plsc.subcore_barrier()

    row0 = s * (N // NS)
    pltpu.sync_copy(agg_sh.at[pl.ds(row0, N // NS)],
                    out_hbm.at[c, pl.ds(row0, N // NS)])


def kernel(x, ei, ea, W_proj, b_proj, eps, We, be, Wm1, bm1, Wm2, bm2,
           gamma, beta):
    src = ei[0].astype(jnp.int32)
    dst = ei[1].astype(jnp.int32)
    zeros = jnp.zeros((N // NS, H), jnp.float32)

    h = _proj(x, W_proj, b_proj.reshape(1, H), 1000)

    for l in range(L):
        e = _proj(ea, We[l], be[l].reshape(1, H), 4000)
        aggp = _agg(e, h, src, dst, zeros)
        scale = (1.0 + eps[l]).reshape(1, 1).astype(jnp.float32)
        h = _node_update(h, aggp[0], aggp[1], scale, Wm1[l],
                         bm1[l].reshape(1, 2 * H), Wm2[l],
                         bm2[l].reshape(1, H), gamma[l].reshape(1, H),
                         beta[l].reshape(1, H), 1000)
    return h


# speedup vs baseline: 2.7470x; 2.7470x over previous
import functools

import jax
import jax.numpy as jnp
from jax import lax
from jax.experimental import pallas as pl
from jax.experimental.pallas import tpu as pltpu
from jax.experimental.pallas import tpu_sc as plsc

N, E, D, ED, H, L = 10000, 320000, 128, 16, 128, 3
LANES = 16
NC, NS = 2, 16
NPAD = 10240
NW = NC * NS
EPT = E // NW
C = 80
ITERS = EPT // C


def _proj_body(x_ref, w_ref, b_ref, o_ref):
    o_ref[...] = (
        jax.lax.dot_general(x_ref[...], w_ref[...], (((1,), (0,)), ((), ())),
                            preferred_element_type=jnp.float32,
                            precision=jax.lax.Precision.HIGHEST)
        + b_ref[...]
    )


def _proj(x, w, b, blk):
    n = x.shape[0]
    return pl.pallas_call(
        _proj_body,
        grid=(n // blk,),
        in_specs=[
            pl.BlockSpec((blk, x.shape[1]), lambda i: (i, jnp.int32(0))),
            pl.BlockSpec(w.shape, lambda i: (jnp.int32(0), jnp.int32(0))),
            pl.BlockSpec((1, b.shape[1]), lambda i: (jnp.int32(0), jnp.int32(0))),
        ],
        out_specs=pl.BlockSpec((blk, w.shape[1]), lambda i: (i, jnp.int32(0))),
        out_shape=jax.ShapeDtypeStruct((n, w.shape[1]), jnp.float32),
    )(x, w, b)


def _node_body(h_ref, a0_ref, a1_ref, scale_ref, w1_ref, b1_ref, w2_ref,
               b2_ref, g_ref, be_ref, o_ref):
    z = scale_ref[0, 0] * h_ref[...] + a0_ref[...] + a1_ref[...]
    t = jax.lax.dot_general(z, w1_ref[...], (((1,), (0,)), ((), ())),
                            preferred_element_type=jnp.float32,
                            precision=jax.lax.Precision.HIGHEST)
    t = jnp.maximum(t + b1_ref[...], 0.0)
    y = jax.lax.dot_general(t, w2_ref[...], (((1,), (0,)), ((), ())),
                            preferred_element_type=jnp.float32,
                            precision=jax.lax.Precision.HIGHEST)
    y = y + b2_ref[...]
    mu = jnp.mean(y, axis=-1, keepdims=True)
    var = jnp.mean((y - mu) ** 2, axis=-1, keepdims=True)
    y = (y - mu) * jax.lax.rsqrt(var + 1e-5) * g_ref[...] + be_ref[...]
    o_ref[...] = jnp.maximum(y, 0.0)


def _node_update(h, a0, a1, scale, w1, b1, w2, b2, gamma, beta, blk):
    n = h.shape[0]
    return pl.pallas_call(
        _node_body,
        grid=(n // blk,),
        in_specs=[
            pl.BlockSpec((blk, H), lambda i: (i, jnp.int32(0))),
            pl.BlockSpec((blk, H), lambda i: (i, jnp.int32(0))),
            pl.BlockSpec((blk, H), lambda i: (i, jnp.int32(0))),
            pl.BlockSpec((1, 1), lambda i: (jnp.int32(0), jnp.int32(0)),
                         memory_space=pltpu.SMEM),
            pl.BlockSpec((H, 2 * H), lambda i: (jnp.int32(0), jnp.int32(0))),
            pl.BlockSpec((1, 2 * H), lambda i: (jnp.int32(0), jnp.int32(0))),
            pl.BlockSpec((2 * H, H), lambda i: (jnp.int32(0), jnp.int32(0))),
            pl.BlockSpec((1, H), lambda i: (jnp.int32(0), jnp.int32(0))),
            pl.BlockSpec((1, H), lambda i: (jnp.int32(0), jnp.int32(0))),
            pl.BlockSpec((1, H), lambda i: (jnp.int32(0), jnp.int32(0))),
        ],
        out_specs=pl.BlockSpec((blk, H), lambda i: (i, jnp.int32(0))),
        out_shape=jax.ShapeDtypeStruct((n, H), jnp.float32),
    )(h, a0, a1, scale, w1, b1, w2, b2, gamma, beta)


@functools.cache
def _build_agg():
    return functools.partial(
        pl.kernel,
        out_type=jax.ShapeDtypeStruct((NC, NPAD, H), jnp.float32),
        mesh=plsc.VectorSubcoreMesh(core_axis_name="c", subcore_axis_name="s",
                                    num_cores=NC, num_subcores=NS),
        scratch_types=[
            pltpu.VMEM((C,), jnp.int32),
            pltpu.VMEM((C,), jnp.int32),
            pltpu.VMEM((C, H), jnp.float32),
            pltpu.VMEM((C, H), jnp.float32),
            pltpu.VMEM_SHARED((NPAD, H), jnp.float32),
            pltpu.SemaphoreType.DMA,
            pltpu.SemaphoreType.DMA,
        ],
    )(_agg_body)


def _agg_body(e_hbm, h_hbm, src_hbm, dst_hbm, zeros_hbm, out_hbm,
              src_v, dst_v, e_v, g_v, agg_sh, sem_e, sem_g):
    c = lax.axis_index("c")
    s = lax.axis_index("s")
    wid = s * NC + c

    pltpu.sync_copy(zeros_hbm, agg_sh.at[pl.ds(s * (NPAD // NS), NPAD // NS)])
    plsc.subcore_barrier()

    def chunk(it, carry):
        base = wid * jnp.int32(EPT) + it * jnp.int32(C)
        pltpu.sync_copy(src_hbm.at[pl.ds(base, C)], src_v)
        pltpu.sync_copy(dst_hbm.at[pl.ds(base, C)], dst_v)
        cp_e = pltpu.async_copy(e_hbm.at[pl.ds(base, C)], e_v, sem_e)
        cp_g = pltpu.async_copy(h_hbm.at[src_v], g_v, sem_g)
        cp_e.wait()
        cp_g.wait()

        def row(i, carry2):
            for j in range(H // LANES):
                sl = pl.ds(j * LANES, LANES)
                g_v[i, sl] = jnp.maximum(g_v[i, sl] + e_v[i, sl], 0.0)
            return carry2

        lax.fori_loop(jnp.int32(0), jnp.int32(C), row, jnp.int32(0))
        pltpu.sync_copy(g_v, agg_sh.at[dst_v], add=True)
        return carry

    lax.fori_loop(jnp.int32(0), jnp.int32(ITERS), chunk, jnp.int32(0))
    plsc.subcore_barrier()

    row0 = s * (NPAD // NS)
    pltpu.sync_copy(agg_sh.at[pl.ds(row0, NPAD // NS)],
                    out_hbm.at[c, pl.ds(row0, NPAD // NS)])


def kernel(x, ei, ea, W_proj, b_proj, eps, We, be, Wm1, bm1, Wm2, bm2,
           gamma, beta):
    src = ei[0].astype(jnp.int32)
    dst = ei[1].astype(jnp.int32)
    zeros = jnp.zeros((NPAD // NS, H), jnp.float32)

    h = _proj(x, W_proj, b_proj.reshape(1, H), 1000)

    for l in range(L):
        e = _proj(ea, We[l], be[l].reshape(1, H), 4000)
        aggp = _build_agg()(e, h, src, dst, zeros)
        scale = (1.0 + eps[l]).reshape(1, 1).astype(jnp.float32)
        h = _node_update(h, aggp[0], aggp[1], scale, Wm1[l],
                         bm1[l].reshape(1, 2 * H), Wm2[l],
                         bm2[l].reshape(1, H), gamma[l].reshape(1, H),
                         beta[l].reshape(1, H), 1000)
    return h
